# baseline (speedup 1.0000x reference)
import functools

import jax
import jax.numpy as jnp
from jax.experimental import pallas as pl
from jax.experimental.pallas import tpu as pltpu

N_PAD = 10240
ROW_BLK = 1024


def _mm_body(x_ref, w_ref, b_ref, o_ref, *, relu):
    acc = jnp.dot(x_ref[...], w_ref[...], preferred_element_type=jnp.float32)
    acc = acc + b_ref[...]
    if relu:
        acc = jnp.maximum(acc, 0.0)
    o_ref[...] = acc


@functools.partial(jax.jit, static_argnames=("relu",))
def _mm(x, w, b, relu=True):
    m, k = x.shape
    n = w.shape[1]
    grid = (m // ROW_BLK,)
    return pl.pallas_call(
        functools.partial(_mm_body, relu=relu),
        grid=grid,
        in_specs=[
            pl.BlockSpec((ROW_BLK, k), lambda i: (i, 0)),
            pl.BlockSpec((k, n), lambda i: (0, 0)),
            pl.BlockSpec((n,), lambda i: (0,)),
        ],
        out_specs=pl.BlockSpec((ROW_BLK, n), lambda i: (i, 0)),
        out_shape=jax.ShapeDtypeStruct((m, n), jnp.float32),
    )(x, w, b)


def _aggregate(h, src, dst, norm, n):
    msg = h[src] * norm[:, None]
    return jnp.zeros((n, h.shape[1]), h.dtype).at[dst].add(msg)


def kernel(x, edge_index, edge_attr, params):
    del edge_attr
    n = x.shape[0]
    p = params
    loop = jnp.arange(n, dtype=edge_index.dtype)
    src = jnp.concatenate([edge_index[0], loop]).astype(jnp.int32)
    dst = jnp.concatenate([edge_index[1], loop]).astype(jnp.int32)
    deg = jnp.zeros((n,), jnp.float32).at[dst].add(1.0)
    dinv = jnp.where(deg > 0, 1.0 / jnp.sqrt(deg), 0.0)
    norm = dinv[src] * dinv[dst]

    xp = jnp.zeros((N_PAD, x.shape[1]), jnp.float32).at[:n].set(x)
    h = _mm(xp, p['lin0_W'], p['lin0_b'])
    h = _mm(h, p['lin1_W'], p['lin1_b'])
    h = _mm(h, p['lin2_W'], p['lin2_b'])

    for i in range(3):
        outs = []
        cur = h
        for j in range(4):
            hw = _mm(cur, p[f'conv{i}_{j}_W'], jnp.zeros_like(p[f'conv{i}_{j}_b']),
                     relu=False)
            agg = _aggregate(hw[:n], src, dst, norm, n)
            cur = jnp.maximum(agg + p[f'conv{i}_{j}_b'], 0.0)
            cur = jnp.zeros((N_PAD, cur.shape[1]), jnp.float32).at[:n].set(cur)
            outs.append(cur)
        cat = jnp.concatenate(outs, axis=-1)
        h = _mm(cat, p[f'jk{i}_W'], p[f'jk{i}_b'])

    out = _mm(h, p['out_W'], p['out_b'], relu=False)
    return out[:n]


# speedup vs baseline: 3.8633x; 3.8633x over previous
import functools

import jax
import jax.numpy as jnp
from jax import lax
from jax.experimental import pallas as pl
from jax.experimental.pallas import tpu as pltpu
from jax.experimental.pallas import tpu_sc as plsc

N = 10000
N_PAD = 10240
ROW_BLK = 1024
DIM_H = 512
QD = 128
E = 160000
E_CHUNK = 128
E_CHUNKS = 1280
E_PAD = E_CHUNKS * E_CHUNK
PAD_NODE = 10200
SUB_ROWS = N_PAD // 16


def _mm_body(x_ref, w_ref, b_ref, o_ref, *, relu):
    acc = jnp.dot(x_ref[...], w_ref[...], preferred_element_type=jnp.float32)
    acc = acc + b_ref[...]
    if relu:
        acc = jnp.maximum(acc, 0.0)
    o_ref[...] = acc


@functools.partial(jax.jit, static_argnames=("relu",))
def _mm(x, w, b, relu=True):
    m, k = x.shape
    n = w.shape[1]
    return pl.pallas_call(
        functools.partial(_mm_body, relu=relu),
        grid=(m // ROW_BLK,),
        in_specs=[
            pl.BlockSpec((ROW_BLK, k), lambda i: (i, 0)),
            pl.BlockSpec((k, n), lambda i: (0, 0)),
            pl.BlockSpec((n,), lambda i: (0,)),
        ],
        out_specs=pl.BlockSpec((ROW_BLK, n), lambda i: (i, 0)),
        out_shape=jax.ShapeDtypeStruct((m, n), jnp.float32),
    )(x, w, b)


def _mm4_body(x0, x1, x2, x3, w_ref, b_ref, o_ref):
    acc = jnp.dot(x0[...], w_ref[0], preferred_element_type=jnp.float32)
    acc += jnp.dot(x1[...], w_ref[1], preferred_element_type=jnp.float32)
    acc += jnp.dot(x2[...], w_ref[2], preferred_element_type=jnp.float32)
    acc += jnp.dot(x3[...], w_ref[3], preferred_element_type=jnp.float32)
    o_ref[...] = jnp.maximum(acc + b_ref[...], 0.0)


@jax.jit
def _mm4(xs, w, b):
    w4 = w.reshape(4, DIM_H, DIM_H)
    return pl.pallas_call(
        _mm4_body,
        grid=(N_PAD // ROW_BLK,),
        in_specs=[pl.BlockSpec((ROW_BLK, DIM_H), lambda i: (i, 0))] * 4
        + [
            pl.BlockSpec((4, DIM_H, DIM_H), lambda i: (0, 0, 0)),
            pl.BlockSpec((DIM_H,), lambda i: (0,)),
        ],
        out_specs=pl.BlockSpec((ROW_BLK, DIM_H), lambda i: (i, 0)),
        out_shape=jax.ShapeDtypeStruct((N_PAD, DIM_H), jnp.float32),
    )(*xs, w4, b)


def _conv_mm_body(x_ref, w_ref, d_ref, o0, o1, o2, o3):
    h = jnp.dot(x_ref[...], w_ref[...], preferred_element_type=jnp.float32)
    h = h * d_ref[...]
    o0[...] = h[:, 0 * QD:1 * QD]
    o1[...] = h[:, 1 * QD:2 * QD]
    o2[...] = h[:, 2 * QD:3 * QD]
    o3[...] = h[:, 3 * QD:4 * QD]


@jax.jit
def _conv_mm(x, w, dinv):
    outs = [jax.ShapeDtypeStruct((N_PAD, QD), jnp.float32)] * 4
    return pl.pallas_call(
        _conv_mm_body,
        grid=(N_PAD // ROW_BLK,),
        in_specs=[
            pl.BlockSpec((ROW_BLK, DIM_H), lambda i: (i, 0)),
            pl.BlockSpec((DIM_H, DIM_H), lambda i: (0, 0)),
            pl.BlockSpec((ROW_BLK, 1), lambda i: (i, 0)),
        ],
        out_specs=[pl.BlockSpec((ROW_BLK, QD), lambda i: (i, 0))] * 4,
        out_shape=outs,
    )(x, w, dinv)


def _post_body(s0, s1, s2, s3, h0, h1, h2, h3, d_ref, b_ref, o_ref):
    s = jnp.concatenate([s0[...], s1[...], s2[...], s3[...]], axis=1)
    h = jnp.concatenate([h0[...], h1[...], h2[...], h3[...]], axis=1)
    o_ref[...] = jnp.maximum(d_ref[...] * (s + h) + b_ref[...], 0.0)


@jax.jit
def _post(ss, hs, dinv, b):
    return pl.pallas_call(
        _post_body,
        grid=(N_PAD // ROW_BLK,),
        in_specs=[pl.BlockSpec((ROW_BLK, QD), lambda i: (i, 0))] * 8
        + [
            pl.BlockSpec((ROW_BLK, 1), lambda i: (i, 0)),
            pl.BlockSpec((DIM_H,), lambda i: (0,)),
        ],
        out_specs=pl.BlockSpec((ROW_BLK, DIM_H), lambda i: (i, 0)),
        out_shape=jax.ShapeDtypeStruct((N_PAD, DIM_H), jnp.float32),
    )(*ss, *hs, dinv, b)


_MESH = plsc.VectorSubcoreMesh(core_axis_name="c", subcore_axis_name="s")


def _zero_rows(rows):
    def zrow(r, _):
        for j in range(8):
            rows[r, pl.ds(j * 16, 16)] = jnp.zeros((16,), jnp.float32)
        return 0
    lax.fori_loop(0, E_CHUNK, zrow, 0)


def _agg_body(h0, h1, h2, h3, src_r, dst_r, o0, o1, o2, o3,
              acc, idx_s, idx_d, rows, sem):
    c = lax.axis_index("c")
    s = lax.axis_index("s")

    def quarter(h_ref, o_ref):
        _zero_rows(rows)

        def zacc(z, _):
            pltpu.sync_copy(rows, acc.at[pl.ds(s * SUB_ROWS + z * 128, 128)])
            return 0
        lax.fori_loop(0, SUB_ROWS // 128, zacc, 0)
        plsc.subcore_barrier()

        def eloop(k, _):
            chunk = s * (E_CHUNKS // 16) + k
            pltpu.sync_copy(src_r.at[chunk], idx_s.at[0])
            pltpu.sync_copy(dst_r.at[chunk], idx_d.at[0])
            pltpu.async_copy(h_ref.at[idx_s.at[0]], rows, sem).wait()
            pltpu.sync_copy(rows, acc.at[idx_d.at[0]], add=True)
            return 0
        lax.fori_loop(0, E_CHUNKS // 16, eloop, 0)
        plsc.subcore_barrier()
        pltpu.sync_copy(acc.at[pl.ds(s * SUB_ROWS, SUB_ROWS)],
                        o_ref.at[pl.ds(s * SUB_ROWS, SUB_ROWS)])
        plsc.subcore_barrier()

    @pl.when(c == 0)
    def _():
        quarter(h0, o0)
        quarter(h1, o1)

    @pl.when(c == 1)
    def _():
        quarter(h2, o2)
        quarter(h3, o3)


@jax.jit
def _agg(hs, src_r, dst_r):
    out = [jax.ShapeDtypeStruct((N_PAD, QD), jnp.float32)] * 4
    f = pl.kernel(
        _agg_body,
        out_type=out,
        mesh=_MESH,
        scratch_types=[
            pltpu.VMEM_SHARED((N_PAD, QD), jnp.float32),
            pltpu.VMEM((1, E_CHUNK), jnp.int32),
            pltpu.VMEM((1, E_CHUNK), jnp.int32),
            pltpu.VMEM((E_CHUNK, QD), jnp.float32),
            pltpu.SemaphoreType.DMA,
        ],
    )
    return f(*hs, src_r, dst_r)


def _deg_body(dst_r, o_ref, acc, idx_d, ones, zbuf):
    c = lax.axis_index("c")
    s = lax.axis_index("s")

    def fill(r, _):
        ones[r, :] = jnp.ones((16,), jnp.float32)
        zbuf[r, :] = jnp.zeros((16,), jnp.float32)
        return 0
    lax.fori_loop(0, E_CHUNK, fill, 0)

    @pl.when(c == 0)
    def _():
        def zacc(z, _):
            pltpu.sync_copy(zbuf, acc.at[pl.ds(s * SUB_ROWS + z * 128, 128)])
            return 0
        lax.fori_loop(0, SUB_ROWS // 128, zacc, 0)
    plsc.subcore_barrier()

    @pl.when(c == 0)
    def _():
        def eloop(k, _):
            chunk = s * (E_CHUNKS // 16) + k
            pltpu.sync_copy(dst_r.at[chunk], idx_d.at[0])
            pltpu.sync_copy(ones, acc.at[idx_d.at[0]], add=True)
            return 0
        lax.fori_loop(0, E_CHUNKS // 16, eloop, 0)
    plsc.subcore_barrier()

    @pl.when(c == 0)
    def _():
        pltpu.sync_copy(acc.at[pl.ds(s * SUB_ROWS, SUB_ROWS)],
                        o_ref.at[pl.ds(s * SUB_ROWS, SUB_ROWS)])


@jax.jit
def _deg(dst_r):
    f = pl.kernel(
        _deg_body,
        out_type=jax.ShapeDtypeStruct((N_PAD, 16), jnp.float32),
        mesh=_MESH,
        scratch_types=[
            pltpu.VMEM_SHARED((N_PAD, 16), jnp.float32),
            pltpu.VMEM((1, E_CHUNK), jnp.int32),
            pltpu.VMEM((E_CHUNK, 16), jnp.float32),
            pltpu.VMEM((E_CHUNK, 16), jnp.float32),
        ],
    )
    return f(dst_r)


def kernel(x, edge_index, edge_attr, params):
    del edge_attr
    p = params
    src = edge_index[0].astype(jnp.int32)
    dst = edge_index[1].astype(jnp.int32)
    pad = jnp.full((E_PAD - E,), PAD_NODE, jnp.int32)
    src_r = jnp.concatenate([src, pad]).reshape(E_CHUNKS, E_CHUNK)
    dst_r = jnp.concatenate([dst, pad]).reshape(E_CHUNKS, E_CHUNK)

    degc = _deg(dst_r)
    dinv = lax.rsqrt(degc[:, 0:1] + 1.0)

    xp = jnp.zeros((N_PAD, x.shape[1]), jnp.float32).at[:N].set(x)
    h = _mm(xp, p['lin0_W'], p['lin0_b'])
    h = _mm(h, p['lin1_W'], p['lin1_b'])
    h = _mm(h, p['lin2_W'], p['lin2_b'])

    for i in range(3):
        outs = []
        cur = h
        for j in range(4):
            hq = _conv_mm(cur, p[f'conv{i}_{j}_W'], dinv)
            sq = _agg(hq, src_r, dst_r)
            cur = _post(sq, hq, dinv, p[f'conv{i}_{j}_b'])
            outs.append(cur)
        h = _mm4(outs, p[f'jk{i}_W'], p[f'jk{i}_b'])

    out = _mm(h, p['out_W'], p['out_b'], relu=False)
    return out[:N]


# speedup vs baseline: 5.1294x; 1.3277x over previous
import functools

import jax
import jax.numpy as jnp
from jax import lax
from jax.experimental import pallas as pl
from jax.experimental.pallas import tpu as pltpu
from jax.experimental.pallas import tpu_sc as plsc

N = 10000
N_PAD = 10240
ROW_BLK = 1024
DIM_H = 512
QD = 128
E = 160000
E_CHUNK = 128
E_CHUNKS = 1280
E_PAD = E_CHUNKS * E_CHUNK
PAD_NODE = 10200
SUB_ROWS = N_PAD // 16


def _mm_body(x_ref, w_ref, b_ref, o_ref, *, relu):
    acc = jnp.dot(x_ref[...], w_ref[...], preferred_element_type=jnp.float32)
    acc = acc + b_ref[...]
    if relu:
        acc = jnp.maximum(acc, 0.0)
    o_ref[...] = acc


@functools.partial(jax.jit, static_argnames=("relu",))
def _mm(x, w, b, relu=True):
    m, k = x.shape
    n = w.shape[1]
    return pl.pallas_call(
        functools.partial(_mm_body, relu=relu),
        grid=(m // ROW_BLK,),
        in_specs=[
            pl.BlockSpec((ROW_BLK, k), lambda i: (i, 0)),
            pl.BlockSpec((k, n), lambda i: (0, 0)),
            pl.BlockSpec((n,), lambda i: (0,)),
        ],
        out_specs=pl.BlockSpec((ROW_BLK, n), lambda i: (i, 0)),
        out_shape=jax.ShapeDtypeStruct((m, n), jnp.float32),
    )(x, w, b)


def _mm4_body(x0, x1, x2, x3, w_ref, b_ref, o_ref):
    acc = jnp.dot(x0[...], w_ref[0], preferred_element_type=jnp.float32)
    acc += jnp.dot(x1[...], w_ref[1], preferred_element_type=jnp.float32)
    acc += jnp.dot(x2[...], w_ref[2], preferred_element_type=jnp.float32)
    acc += jnp.dot(x3[...], w_ref[3], preferred_element_type=jnp.float32)
    o_ref[...] = jnp.maximum(acc + b_ref[...], 0.0)


@jax.jit
def _mm4(xs, w, b):
    w4 = w.reshape(4, DIM_H, DIM_H)
    return pl.pallas_call(
        _mm4_body,
        grid=(N_PAD // ROW_BLK,),
        in_specs=[pl.BlockSpec((ROW_BLK, DIM_H), lambda i: (i, 0))] * 4
        + [
            pl.BlockSpec((4, DIM_H, DIM_H), lambda i: (0, 0, 0)),
            pl.BlockSpec((DIM_H,), lambda i: (0,)),
        ],
        out_specs=pl.BlockSpec((ROW_BLK, DIM_H), lambda i: (i, 0)),
        out_shape=jax.ShapeDtypeStruct((N_PAD, DIM_H), jnp.float32),
    )(*xs, w4, b)


def _conv_mm_body(x_ref, w_ref, d_ref, o0, o1, o2, o3):
    h = jnp.dot(x_ref[...], w_ref[...], preferred_element_type=jnp.float32)
    h = h * d_ref[...]
    o0[...] = h[:, 0 * QD:1 * QD]
    o1[...] = h[:, 1 * QD:2 * QD]
    o2[...] = h[:, 2 * QD:3 * QD]
    o3[...] = h[:, 3 * QD:4 * QD]


@jax.jit
def _conv_mm(x, w, dinv):
    outs = [jax.ShapeDtypeStruct((N_PAD, QD), jnp.float32)] * 4
    return pl.pallas_call(
        _conv_mm_body,
        grid=(N_PAD // ROW_BLK,),
        in_specs=[
            pl.BlockSpec((ROW_BLK, DIM_H), lambda i: (i, 0)),
            pl.BlockSpec((DIM_H, DIM_H), lambda i: (0, 0)),
            pl.BlockSpec((ROW_BLK, 1), lambda i: (i, 0)),
        ],
        out_specs=[pl.BlockSpec((ROW_BLK, QD), lambda i: (i, 0))] * 4,
        out_shape=outs,
    )(x, w, dinv)


def _post_body(s0, s1, s2, s3, h0, h1, h2, h3, d_ref, b_ref, o_ref):
    s = jnp.concatenate([s0[...], s1[...], s2[...], s3[...]], axis=1)
    h = jnp.concatenate([h0[...], h1[...], h2[...], h3[...]], axis=1)
    o_ref[...] = jnp.maximum(d_ref[...] * (s + h) + b_ref[...], 0.0)


@jax.jit
def _post(ss, hs, dinv, b):
    return pl.pallas_call(
        _post_body,
        grid=(N_PAD // ROW_BLK,),
        in_specs=[pl.BlockSpec((ROW_BLK, QD), lambda i: (i, 0))] * 8
        + [
            pl.BlockSpec((ROW_BLK, 1), lambda i: (i, 0)),
            pl.BlockSpec((DIM_H,), lambda i: (0,)),
        ],
        out_specs=pl.BlockSpec((ROW_BLK, DIM_H), lambda i: (i, 0)),
        out_shape=jax.ShapeDtypeStruct((N_PAD, DIM_H), jnp.float32),
    )(*ss, *hs, dinv, b)


_MESH = plsc.VectorSubcoreMesh(core_axis_name="c", subcore_axis_name="s")


CPT = E_CHUNKS // 16
ZROWS = 64


def _fill_zeros(buf, nrows):
    def zrow(r, _):
        for j in range(8):
            buf[r, pl.ds(j * 16, 16)] = jnp.zeros((16,), jnp.float32)
        return 0
    lax.fori_loop(0, nrows, zrow, 0)


def _agg_body(h0, h1, h2, h3, src_r, dst_r, o0, o1, o2, o3,
              acc, src_t, dst_t, rows, zbuf, gsem, ssem, issrc, isdst):
    c = lax.axis_index("c")
    s = lax.axis_index("s")
    base = s * CPT
    _fill_zeros(zbuf, ZROWS)

    def quarter(h_ref, o_ref):
        def zacc(z, _):
            pltpu.sync_copy(zbuf,
                            acc.at[pl.ds(s * SUB_ROWS + z * ZROWS, ZROWS)])
            return 0
        lax.fori_loop(0, SUB_ROWS // ZROWS, zacc, 0)
        plsc.subcore_barrier()

        pltpu.sync_copy(src_r.at[pl.ds(base, 2)], src_t.at[pl.ds(0, 2)])
        pltpu.sync_copy(dst_r.at[pl.ds(base, 2)], dst_t.at[pl.ds(0, 2)])

        def chunk_fn(k, _):
            b = lax.rem(k, 2)
            q = lax.rem(k, 4)
            qn = lax.rem(k + 2, 4)

            @pl.when(k >= 2)
            def _():
                pltpu.make_async_copy(h_ref.at[pl.ds(0, E_CHUNK)],
                                      rows.at[b], ssem.at[b]).wait()

            @pl.when(k < CPT - 2)
            def _():
                pltpu.async_copy(src_r.at[base + k + 2], src_t.at[qn],
                                 issrc.at[qn])
                pltpu.async_copy(dst_r.at[base + k + 2], dst_t.at[qn],
                                 isdst.at[qn])

            @pl.when(k >= 2)
            def _():
                pltpu.make_async_copy(src_r.at[0], src_t.at[q],
                                      issrc.at[q]).wait()
                pltpu.make_async_copy(dst_r.at[0], dst_t.at[q],
                                      isdst.at[q]).wait()

            pltpu.async_copy(h_ref.at[src_t.at[q]], rows.at[b],
                             gsem.at[b]).wait()
            pltpu.async_copy(rows.at[b], acc.at[dst_t.at[q]],
                             ssem.at[b], add=True)
            return 0
        lax.fori_loop(0, CPT, chunk_fn, 0)
        for b in range(2):
            pltpu.make_async_copy(h_ref.at[pl.ds(0, E_CHUNK)],
                                  rows.at[b], ssem.at[b]).wait()
        plsc.subcore_barrier()
        pltpu.sync_copy(acc.at[pl.ds(s * SUB_ROWS, SUB_ROWS)],
                        o_ref.at[pl.ds(s * SUB_ROWS, SUB_ROWS)])
        plsc.subcore_barrier()

    @pl.when(c == 0)
    def _():
        quarter(h0, o0)
        quarter(h1, o1)

    @pl.when(c == 1)
    def _():
        quarter(h2, o2)
        quarter(h3, o3)


@jax.jit
def _agg(hs, src_r, dst_r):
    out = [jax.ShapeDtypeStruct((N_PAD, QD), jnp.float32)] * 4
    f = pl.kernel(
        _agg_body,
        out_type=out,
        mesh=_MESH,
        scratch_types=[
            pltpu.VMEM_SHARED((N_PAD, QD), jnp.float32),
            pltpu.VMEM((4, E_CHUNK), jnp.int32),
            pltpu.VMEM((4, E_CHUNK), jnp.int32),
            pltpu.VMEM((2, E_CHUNK, QD), jnp.float32),
            pltpu.VMEM((ZROWS, QD), jnp.float32),
            pltpu.SemaphoreType.DMA((2,)),
            pltpu.SemaphoreType.DMA((2,)),
            pltpu.SemaphoreType.DMA((4,)),
            pltpu.SemaphoreType.DMA((4,)),
        ],
    )
    return f(*hs, src_r, dst_r)


def _deg_body(dst_r, o_ref, acc, idx_d, ones, zbuf):
    c = lax.axis_index("c")
    s = lax.axis_index("s")

    def fill(r, _):
        ones[r, :] = jnp.ones((16,), jnp.float32)
        zbuf[r, :] = jnp.zeros((16,), jnp.float32)
        return 0
    lax.fori_loop(0, E_CHUNK, fill, 0)

    @pl.when(c == 0)
    def _():
        def zacc(z, _):
            pltpu.sync_copy(zbuf, acc.at[pl.ds(s * SUB_ROWS + z * 128, 128)])
            return 0
        lax.fori_loop(0, SUB_ROWS // 128, zacc, 0)
    plsc.subcore_barrier()

    @pl.when(c == 0)
    def _():
        def eloop(k, _):
            chunk = s * (E_CHUNKS // 16) + k
            pltpu.sync_copy(dst_r.at[chunk], idx_d.at[0])
            pltpu.sync_copy(ones, acc.at[idx_d.at[0]], add=True)
            return 0
        lax.fori_loop(0, E_CHUNKS // 16, eloop, 0)
    plsc.subcore_barrier()

    @pl.when(c == 0)
    def _():
        pltpu.sync_copy(acc.at[pl.ds(s * SUB_ROWS, SUB_ROWS)],
                        o_ref.at[pl.ds(s * SUB_ROWS, SUB_ROWS)])


@jax.jit
def _deg(dst_r):
    f = pl.kernel(
        _deg_body,
        out_type=jax.ShapeDtypeStruct((N_PAD, 16), jnp.float32),
        mesh=_MESH,
        scratch_types=[
            pltpu.VMEM_SHARED((N_PAD, 16), jnp.float32),
            pltpu.VMEM((1, E_CHUNK), jnp.int32),
            pltpu.VMEM((E_CHUNK, 16), jnp.float32),
            pltpu.VMEM((E_CHUNK, 16), jnp.float32),
        ],
    )
    return f(dst_r)


def kernel(x, edge_index, edge_attr, params):
    del edge_attr
    p = params
    src = edge_index[0].astype(jnp.int32)
    dst = edge_index[1].astype(jnp.int32)
    pad = jnp.full((E_PAD - E,), PAD_NODE, jnp.int32)
    src_r = jnp.concatenate([src, pad]).reshape(E_CHUNKS, E_CHUNK)
    dst_r = jnp.concatenate([dst, pad]).reshape(E_CHUNKS, E_CHUNK)

    degc = _deg(dst_r)
    dinv = lax.rsqrt(degc[:, 0:1] + 1.0)

    xp = jnp.zeros((N_PAD, x.shape[1]), jnp.float32).at[:N].set(x)
    h = _mm(xp, p['lin0_W'], p['lin0_b'])
    h = _mm(h, p['lin1_W'], p['lin1_b'])
    h = _mm(h, p['lin2_W'], p['lin2_b'])

    for i in range(3):
        outs = []
        cur = h
        for j in range(4):
            hq = _conv_mm(cur, p[f'conv{i}_{j}_W'], dinv)
            sq = _agg(hq, src_r, dst_r)
            cur = _post(sq, hq, dinv, p[f'conv{i}_{j}_b'])
            outs.append(cur)
        h = _mm4(outs, p[f'jk{i}_W'], p[f'jk{i}_b'])

    out = _mm(h, p['out_W'], p['out_b'], relu=False)
    return out[:N]
